# speedup vs baseline: 13.8921x; 13.8921x over previous
import jax
import jax.numpy as jnp
from jax.experimental import pallas as pl
from jax.experimental.pallas import tpu as pltpu


def _dgsc_kernel(x_ref, adjp_ref, adjpT_ref, x22_ref, wch_ref, wc3t_ref,
                 bch_ref, bcv_ref, out_ref):
    b = pl.program_id(0)
    nn = adjp_ref.shape[0]
    t = out_ref.shape[2]

    dims2d = (((1,), (0,)), ((), ()))

    xb = x_ref[0]
    wd = wch_ref[0] - wch_ref[2]
    y = jax.lax.dot_general(wd, xb, dims2d,
                            preferred_element_type=jnp.float32)
    acc = None
    for n in range(nn):
        part = jax.lax.dot_general(wc3t_ref[n], y[:, n * t:(n + 1) * t], dims2d,
                                   preferred_element_type=jnp.float32)
        acc = part if acc is None else acc + part

    a_oi = jnp.sum(wc3t_ref[...], axis=0)
    const = jax.lax.dot_general(a_oi, bch_ref[...], (((1,), (1,)), ((), ())),
                                preferred_element_type=jnp.float32)
    acc = acc + const + bcv_ref[...]

    adj = 0.5 * (jax.nn.sigmoid(adjp_ref[...]) + jax.nn.sigmoid(adjpT_ref[...]))
    r = jax.lax.broadcasted_iota(jnp.int32, (nn, nn), 0)
    c = jax.lax.broadcasted_iota(jnp.int32, (nn, nn), 1)
    adj = jnp.where(r == c, 0.0, adj)
    deg = jnp.sum(adj, axis=1, keepdims=True)
    degt = jnp.sum(adj, axis=0, keepdims=True)
    dis = jnp.where(deg > 0, jax.lax.rsqrt(deg), 0.0)
    dist = jnp.where(degt > 0, jax.lax.rsqrt(degt), 0.0)
    s = -(dis * adj * dist)

    p1 = jax.lax.dot_general(s, x22_ref[...], dims2d,
                             preferred_element_type=jnp.float32)
    p2 = jax.lax.dot_general(s, p1, dims2d,
                             preferred_element_type=jnp.float32)
    corr = (jax.lax.dot_general(p1, wch_ref[1], (((1,), (1,)), ((), ())),
                                preferred_element_type=jnp.float32)
            + 2.0 * jax.lax.dot_general(p2, wch_ref[2], (((1,), (1,)), ((), ())),
                                        preferred_element_type=jnp.float32))
    prod = wc3t_ref[...] * corr[:, None, :]
    delta = jnp.sum(jnp.sum(prod, axis=0), axis=1, keepdims=True)

    tcol = jax.lax.broadcasted_iota(jnp.int32, acc.shape, 1)
    acc = acc + jnp.where((tcol == 0) & (b == 0), delta, 0.0)

    out_ref[0] = acc


def kernel(x, adj_param, W_cheb, b_cheb, W_conv, b_conv):
    B, CIN, NN, T = x.shape
    K, COUT, _ = W_cheb.shape

    xr = x.reshape(B, CIN, NN * T)
    x22 = x[0, :, :, 0].T
    adjp = adj_param
    adjpT = adj_param.T
    wc3t = jnp.transpose(W_conv[:, :, :, 0], (2, 0, 1))
    bch = b_cheb.reshape(1, COUT)
    bcv = b_conv.reshape(COUT, 1)

    out = pl.pallas_call(
        _dgsc_kernel,
        grid=(B,),
        in_specs=[
            pl.BlockSpec((1, CIN, NN * T), lambda b: (b, 0, 0)),
            pl.BlockSpec((NN, NN), lambda b: (0, 0)),
            pl.BlockSpec((NN, NN), lambda b: (0, 0)),
            pl.BlockSpec((NN, CIN), lambda b: (0, 0)),
            pl.BlockSpec((K, COUT, CIN), lambda b: (0, 0, 0)),
            pl.BlockSpec((NN, COUT, COUT), lambda b: (0, 0, 0)),
            pl.BlockSpec((1, COUT), lambda b: (0, 0)),
            pl.BlockSpec((COUT, 1), lambda b: (0, 0)),
        ],
        out_specs=pl.BlockSpec((1, COUT, T), lambda b: (b, 0, 0)),
        out_shape=jax.ShapeDtypeStruct((B, COUT, T), jnp.float32),
        compiler_params=pltpu.CompilerParams(
            dimension_semantics=("parallel",),
        ),
    )(xr, adjp, adjpT, x22, W_cheb, wc3t, bch, bcv)

    return out[:, :, None, :]
